# baseline (speedup 1.0000x reference)
import functools

import jax
import jax.numpy as jnp
from jax import lax
from jax.experimental import pallas as pl
from jax.experimental.pallas import tpu as pltpu
from jax.experimental.pallas import tpu_sc as plsc

LANES = 16
NUM_CORES = 2
NUM_SUBCORES = 16
NW = NUM_CORES * NUM_SUBCORES


def kernel(user, item_seq, item_seq_len, user_table, item_table, T):
    B = user.shape[0]
    L = item_seq.shape[1]
    D = T.shape[0]
    bpw = B // NW

    item_seq_flat = item_seq.reshape(-1)

    mesh = plsc.VectorSubcoreMesh(core_axis_name="c", subcore_axis_name="s")

    @functools.partial(
        pl.kernel,
        mesh=mesh,
        compiler_params=pltpu.CompilerParams(use_tc_tiling_on_sc=False),
        out_type=jax.ShapeDtypeStruct((B, D), jnp.float32),
        scratch_types=[
            pltpu.VMEM((bpw,), jnp.int32),
            pltpu.VMEM((bpw,), jnp.int32),
            pltpu.VMEM((bpw,), jnp.int32),
            pltpu.VMEM((bpw,), jnp.int32),
            pltpu.VMEM((bpw, D), jnp.float32),
            pltpu.VMEM((bpw, D), jnp.float32),
            pltpu.VMEM((D,), jnp.float32),
            pltpu.SemaphoreType.DMA,
            pltpu.SemaphoreType.DMA,
            pltpu.SemaphoreType.DMA,
        ],
    )
    def run(user_hbm, iseq_hbm, len_hbm, ut_hbm, it_hbm, t_hbm, out_hbm,
            uid_v, len_v, fidx_v, last_v, u_v, i_v, t_v, sem_u, sem_f, sem_i):
        wid = lax.axis_index("s") * NUM_CORES + lax.axis_index("c")
        base = wid * bpw

        pltpu.sync_copy(user_hbm.at[pl.ds(base, bpw)], uid_v)
        pltpu.sync_copy(len_hbm.at[pl.ds(base, bpw)], len_v)
        pltpu.sync_copy(t_hbm, t_v)

        u_cp = pltpu.async_copy(ut_hbm.at[uid_v], u_v, sem_u)

        lane = lax.iota(jnp.int32, LANES)

        def idx_body(i, _):
            l = len_v[pl.ds(i * LANES, LANES)]
            row = base + i * LANES + lane
            fidx_v[pl.ds(i * LANES, LANES)] = row * L + l - 1
            return 0

        lax.fori_loop(0, bpw // LANES, idx_body, 0)

        pltpu.async_copy(iseq_hbm.at[fidx_v], last_v, sem_f).wait()
        i_cp = pltpu.async_copy(it_hbm.at[last_v], i_v, sem_i)
        u_cp.wait()
        i_cp.wait()

        t0 = t_v[pl.ds(0, LANES)]
        t1 = t_v[pl.ds(LANES, LANES)]

        def add_body(r, _):
            u0 = u_v[r, pl.ds(0, LANES)]
            u1 = u_v[r, pl.ds(LANES, LANES)]
            v0 = i_v[r, pl.ds(0, LANES)]
            v1 = i_v[r, pl.ds(LANES, LANES)]
            u_v[r, pl.ds(0, LANES)] = u0 + v0 + t0
            u_v[r, pl.ds(LANES, LANES)] = u1 + v1 + t1
            return 0

        lax.fori_loop(0, bpw, add_body, 0)

        pltpu.sync_copy(u_v, out_hbm.at[pl.ds(base, bpw)])

    return run(user, item_seq_flat, item_seq_len, user_table, item_table, T)


# speedup vs baseline: 3.9009x; 3.9009x over previous
import functools

import jax
import jax.numpy as jnp
from jax import lax
from jax.experimental import pallas as pl
from jax.experimental.pallas import tpu as pltpu
from jax.experimental.pallas import tpu_sc as plsc

LANES = 16
NUM_CORES = 2
NUM_SUBCORES = 16
NW = NUM_CORES * NUM_SUBCORES
TILE = 128
GRP = 4
PAD = LANES - GRP


def kernel(user, item_seq, item_seq_len, user_table, item_table, T):
    B = user.shape[0]
    Lseq = item_seq.shape[1]
    D = T.shape[0]
    V = user_table.shape[0]
    SUB = 8
    AG = D // SUB
    bpw = B // NW
    ngrp = bpw // GRP

    ut3 = user_table.T.reshape(AG, SUB, V)
    it3 = item_table.T.reshape(AG, SUB, V)
    iseq_t = item_seq.T

    mesh = plsc.VectorSubcoreMesh(core_axis_name="c", subcore_axis_name="s")

    blk_shape = (AG, SUB, TILE)
    nslots = 2 * GRP

    @functools.partial(
        pl.kernel,
        mesh=mesh,
        compiler_params=pltpu.CompilerParams(needs_layout_passes=False),
        out_type=jax.ShapeDtypeStruct((D, B), jnp.float32),
        scratch_types=[
            pltpu.VMEM((bpw + PAD,), jnp.int32),
            pltpu.VMEM((bpw,), jnp.int32),
            pltpu.VMEM((Lseq, bpw), jnp.int32),
            pltpu.VMEM((D,), jnp.float32),
            pltpu.VMEM((bpw + PAD,), jnp.int32),
            pltpu.VMEM((D, bpw), jnp.float32),
            [pltpu.VMEM(blk_shape, jnp.float32) for _ in range(nslots)],
            [pltpu.VMEM(blk_shape, jnp.float32) for _ in range(nslots)],
            [pltpu.SemaphoreType.DMA for _ in range(nslots)],
            [pltpu.SemaphoreType.DMA for _ in range(nslots)],
        ],
    )
    def run(user_hbm, iseq_hbm, len_hbm, ut_hbm, it_hbm, t_hbm, out_hbm,
            uid_v, len_v, win_v, t_v, lid_v, out_v,
            ubufs, ibufs, usems, isems):
        wid = lax.axis_index("s") * NUM_CORES + lax.axis_index("c")
        base = wid * bpw

        pltpu.sync_copy(user_hbm.at[pl.ds(base, bpw)], uid_v.at[pl.ds(0, bpw)])
        pltpu.sync_copy(len_hbm.at[pl.ds(base, bpw)], len_v)
        pltpu.sync_copy(iseq_hbm.at[:, pl.ds(base, bpw)], win_v)
        pltpu.sync_copy(t_hbm, t_v)

        lane = lax.iota(jnp.int32, LANES)
        for g in range(bpw // LANES):
            sl = pl.ds(g * LANES, LANES)
            lenm1 = len_v[sl] - 1
            lid_v[sl] = plsc.load_gather(win_v, [lenm1, g * LANES + lane])

        t0 = t_v[pl.ds(0, LANES)]
        t1 = t_v[pl.ds(LANES, LANES)]
        a_lo = lane >> 3
        d_idx = lane & 7
        a_hi = a_lo + 2
        lane_row = lane * bpw
        zero16 = jnp.zeros((LANES,), jnp.int32)

        def start_group(g, slot_off):
            uvec = uid_v[pl.ds(g * GRP, LANES)]
            lvec = lid_v[pl.ds(g * GRP, LANES)]
            for k in range(GRP):
                r = uvec[k]
                q = lvec[k]
                roff = pl.multiple_of((r >> 7) * TILE, TILE)
                qoff = pl.multiple_of((q >> 7) * TILE, TILE)
                pltpu.async_copy(ut_hbm.at[:, :, pl.ds(roff, TILE)],
                                 ubufs[slot_off + k], usems[slot_off + k])
                pltpu.async_copy(it_hbm.at[:, :, pl.ds(qoff, TILE)],
                                 ibufs[slot_off + k], isems[slot_off + k])

        def process_group(g, slot_off):
            uvec = uid_v[pl.ds(g * GRP, LANES)]
            lvec = lid_v[pl.ds(g * GRP, LANES)]
            for k in range(GRP):
                pltpu.make_async_copy(ut_hbm.at[:, :, pl.ds(0, TILE)],
                                      ubufs[slot_off + k],
                                      usems[slot_off + k]).wait()
                pltpu.make_async_copy(it_hbm.at[:, :, pl.ds(0, TILE)],
                                      ibufs[slot_off + k],
                                      isems[slot_off + k]).wait()
                re = zero16 + (uvec[k] & 127)
                qe = zero16 + (lvec[k] & 127)
                u0 = plsc.load_gather(ubufs[slot_off + k], [a_lo, d_idx, re])
                u1 = plsc.load_gather(ubufs[slot_off + k], [a_hi, d_idx, re])
                v0 = plsc.load_gather(ibufs[slot_off + k], [a_lo, d_idx, qe])
                v1 = plsc.load_gather(ibufs[slot_off + k], [a_hi, d_idx, qe])
                i = g * GRP + k
                plsc.store_scatter(out_v, [lane, zero16 + i], u0 + v0 + t0)
                plsc.store_scatter(out_v, [lane + LANES, zero16 + i],
                                   u1 + v1 + t1)

        start_group(0, 0)
        start_group(1, GRP)

        def body(m, _):
            g = m * 2
            process_group(g, 0)

            @pl.when(m < ngrp // 2 - 1)
            def _():
                start_group(g + 2, 0)

            process_group(g + 1, GRP)

            @pl.when(m < ngrp // 2 - 1)
            def _():
                start_group(g + 3, GRP)

            return 0

        lax.fori_loop(0, ngrp // 2, body, 0)

        pltpu.sync_copy(out_v, out_hbm.at[:, pl.ds(base, bpw)])

    out_t = run(user, iseq_t, item_seq_len, ut3, it3, T)
    return out_t.T
